# baseline (speedup 1.0000x reference)
import functools

import jax
import jax.numpy as jnp
from jax import lax
from jax.experimental import pallas as pl
from jax.experimental.pallas import tpu as pltpu
from jax.experimental.pallas import tpu_sc as plsc

_NC = 2
_NS = 16
_NW = _NC * _NS
_L = 16


@functools.lru_cache(maxsize=None)
def _make_hist_kernel(E, NPAD, RH):
    epw = E // _NW
    TBL = RH * NPAD
    mesh = plsc.VectorSubcoreMesh(
        core_axis_name="c", subcore_axis_name="s", num_cores=_NC
    )

    @functools.partial(
        pl.kernel,
        out_type=jax.ShapeDtypeStruct((_NW, TBL), jnp.int32),
        mesh=mesh,
        compiler_params=pltpu.CompilerParams(needs_layout_passes=False),
        scratch_types=[
            pltpu.VMEM((epw,), jnp.int32),
            pltpu.VMEM((epw,), jnp.int32),
            pltpu.VMEM((TBL,), jnp.int32),
            pltpu.SemaphoreType.DMA,
        ],
    )
    def hist(ei_hbm, typ_hbm, out_hbm, src_v, typ_v, tbl_v, sem):
        cid = lax.axis_index("c")
        sid = lax.axis_index("s")
        wid = sid * _NC + cid
        base = wid * epw
        cp1 = pltpu.async_copy(ei_hbm.at[pl.ds(base, epw)], src_v, sem)
        cp2 = pltpu.async_copy(typ_hbm.at[pl.ds(base, epw)], typ_v, sem)

        zero = jnp.zeros((_L,), jnp.int32)
        ZU = 16

        def zero_body(i, carry):
            for j in range(ZU):
                tbl_v[pl.ds((i * ZU + j) * _L, _L)] = zero
            return carry

        lax.fori_loop(0, TBL // (_L * ZU), zero_body, 0)

        cp1.wait()
        cp2.wait()

        EU = 125

        def edge_body(i, carry):
            for j in range(EU):
                off = (i * EU + j) * _L
                s = src_v[pl.ds(off, _L)]
                t = typ_v[pl.ds(off, _L)]
                idx = (t >> 1) * NPAD + s
                odd = t & 1
                plsc.addupdate_scatter(tbl_v, [idx], (odd << 16) + (1 - odd))
            return carry

        lax.fori_loop(0, epw // (_L * EU), edge_body, 0)
        pltpu.sync_copy(tbl_v, out_hbm.at[wid])

    return hist


@functools.lru_cache(maxsize=None)
def _make_combine_kernel(N, NPAD, D, R):
    RP = 8

    def body(p_ref, x_ref, w_ref, wl_ref, b_ref, o_ref):
        p = p_ref[...]
        low = jnp.sum(p & 0xFFFF, axis=0)
        high = jnp.sum(p >> 16, axis=0)
        rows = []
        for j in range(R // 2):
            rows.append(low[j : j + 1])
            rows.append(high[j : j + 1])
        rows.append(jnp.zeros((RP - R, NPAD), jnp.int32))
        cc = jnp.concatenate(rows, axis=0)[:, :N].astype(jnp.float32)
        s = jnp.dot(cc, x_ref[...], precision=lax.Precision.DEFAULT)
        xsum = jnp.sum(x_ref[...], axis=0, keepdims=True)
        row = lax.broadcasted_iota(jnp.int32, (RP, D), 0)
        s = s + jnp.where(row == R, xsum, 0.0)
        g = jnp.zeros((RP, D), jnp.float32)
        for k in range(R):
            sk = jnp.where(row == k, s, 0.0)
            g = g + jnp.dot(sk, w_ref[k], precision=lax.Precision.DEFAULT)
        sl = jnp.where(row == R, s, 0.0)
        g = g + jnp.dot(sl, wl_ref[...], precision=lax.Precision.DEFAULT)
        out = jnp.sum(g, axis=0, keepdims=True) * (1.0 / N) + b_ref[...]
        o_ref[...] = out

    return pl.pallas_call(
        body,
        out_shape=jax.ShapeDtypeStruct((1, D), jnp.float32),
    )


def kernel(x, edge_index, edge_type, W, W_loop, bias):
    N, D = x.shape
    R = W.shape[0]
    E = edge_type.shape[0]
    NPAD = ((N + 127) // 128) * 128
    RH = R // 2

    ei_flat = edge_index.reshape(2 * E)
    hist = _make_hist_kernel(E, NPAD, RH)
    partials = hist(ei_flat, edge_type)
    partials = partials.reshape(_NW, RH, NPAD)

    combine = _make_combine_kernel(N, NPAD, D, R)
    return combine(partials, x, W, W_loop, bias[None])


# speedup vs baseline: 1.0274x; 1.0274x over previous
import functools

import jax
import jax.numpy as jnp
from jax import lax
from jax.experimental import pallas as pl
from jax.experimental.pallas import tpu as pltpu
from jax.experimental.pallas import tpu_sc as plsc

_NC = 2
_NS = 16
_NW = _NC * _NS
_L = 16


@functools.lru_cache(maxsize=None)
def _make_hist_kernel(E, NPAD, RH):
    epw = E // _NW
    TBL = RH * NPAD
    mesh = plsc.VectorSubcoreMesh(
        core_axis_name="c", subcore_axis_name="s", num_cores=_NC
    )

    @functools.partial(
        pl.kernel,
        out_type=jax.ShapeDtypeStruct((_NW, TBL), jnp.int32),
        mesh=mesh,
        compiler_params=pltpu.CompilerParams(needs_layout_passes=False),
        scratch_types=[
            pltpu.VMEM((epw,), jnp.int32),
            pltpu.VMEM((epw,), jnp.int32),
            pltpu.VMEM((TBL,), jnp.int32),
            pltpu.SemaphoreType.DMA,
        ],
    )
    def hist(ei_hbm, typ_hbm, out_hbm, src_v, typ_v, tbl_v, sem):
        cid = lax.axis_index("c")
        sid = lax.axis_index("s")
        wid = sid * _NC + cid
        base = wid * epw
        cp1 = pltpu.async_copy(ei_hbm.at[pl.ds(base, epw)], src_v, sem)
        cp2 = pltpu.async_copy(typ_hbm.at[pl.ds(base, epw)], typ_v, sem)

        zero = jnp.zeros((_L,), jnp.int32)
        ZU = 16

        def zero_body(i, carry):
            for j in range(ZU):
                tbl_v[pl.ds((i * ZU + j) * _L, _L)] = zero
            return carry

        lax.fori_loop(0, TBL // (_L * ZU), zero_body, 0)

        cp1.wait()
        cp2.wait()

        EU = 25

        def edge_body(i, carry):
            for j in range(EU):
                off = (i * EU + j) * _L
                s = src_v[pl.ds(off, _L)]
                t = typ_v[pl.ds(off, _L)]
                idx = (t >> 1) * NPAD + s
                odd = t & 1
                plsc.addupdate_scatter(tbl_v, [idx], (odd << 16) + (1 - odd))
            return carry

        lax.fori_loop(0, epw // (_L * EU), edge_body, 0)
        pltpu.sync_copy(tbl_v, out_hbm.at[wid])

    return hist


@functools.lru_cache(maxsize=None)
def _make_combine_kernel(N, NPAD, D, R):
    RP = 8

    def body(p_ref, x_ref, w_ref, wl_ref, b_ref, o_ref):
        p = p_ref[...]
        low = jnp.sum(p & 0xFFFF, axis=0)
        high = jnp.sum(p >> 16, axis=0)
        rows = []
        for j in range(R // 2):
            rows.append(low[j : j + 1])
            rows.append(high[j : j + 1])
        rows.append(jnp.zeros((RP - R, NPAD), jnp.int32))
        cc = jnp.concatenate(rows, axis=0)[:, :N].astype(jnp.float32)
        s = jnp.dot(cc, x_ref[...], precision=lax.Precision.DEFAULT)
        xsum = jnp.sum(x_ref[...], axis=0, keepdims=True)
        row = lax.broadcasted_iota(jnp.int32, (RP, D), 0)
        s = s + jnp.where(row == R, xsum, 0.0)
        g = jnp.zeros((RP, D), jnp.float32)
        for k in range(R):
            sk = jnp.where(row == k, s, 0.0)
            g = g + jnp.dot(sk, w_ref[k], precision=lax.Precision.DEFAULT)
        sl = jnp.where(row == R, s, 0.0)
        g = g + jnp.dot(sl, wl_ref[...], precision=lax.Precision.DEFAULT)
        out = jnp.sum(g, axis=0, keepdims=True) * (1.0 / N) + b_ref[...]
        o_ref[...] = out

    return pl.pallas_call(
        body,
        out_shape=jax.ShapeDtypeStruct((1, D), jnp.float32),
    )


def kernel(x, edge_index, edge_type, W, W_loop, bias):
    N, D = x.shape
    R = W.shape[0]
    E = edge_type.shape[0]
    NPAD = ((N + 127) // 128) * 128
    RH = R // 2

    ei_flat = edge_index.reshape(2 * E)
    hist = _make_hist_kernel(E, NPAD, RH)
    partials = hist(ei_flat, edge_type)
    partials = partials.reshape(_NW, RH, NPAD)

    combine = _make_combine_kernel(N, NPAD, D, R)
    return combine(partials, x, W, W_loop, bias[None])
